# Initial kernel scaffold; baseline (speedup 1.0000x reference)
import jax
import jax.numpy as jnp
from jax.experimental import pallas as pl


def kernel(pred_logits, gt_label, gt_score):
    raise NotImplementedError("write your pallas kernel here")


# speedup vs baseline: 2.8504x; 2.8504x over previous
import functools

import jax
import jax.numpy as jnp
from jax import lax
from jax.experimental import pallas as pl


def _qfl_block(pred_ref, gts_ref, label_ref, out_ref):
    x = pred_ref[...]
    gt = gts_ref[...]
    lab = label_ref[...]
    B, C = x.shape

    cols = lax.broadcasted_iota(jnp.int32, (B, C), 1)
    onehot = cols == lab
    gs = jnp.sum(jnp.where(onehot, gt, 0.0), axis=1, keepdims=True)

    s = jax.nn.sigmoid(x)
    softplus_neg = jnp.log1p(jnp.exp(-jnp.abs(x)))
    relu_x = jnp.maximum(x, 0.0)
    ce = (relu_x + softplus_neg) * (s * s)

    sf = gs - s
    vals = (relu_x - x * gs + softplus_neg) * (sf * sf)

    mask = (lab >= 0) & (lab < C)
    out_ref[...] = jnp.where(onehot & mask, vals, ce)


@jax.jit
def kernel(pred_logits, gt_label, gt_score):
    N, C = pred_logits.shape
    B = 2000
    grid = (N // B,)
    lab2d = gt_label.astype(jnp.int32).reshape(N, 1)
    return pl.pallas_call(
        _qfl_block,
        grid=grid,
        in_specs=[
            pl.BlockSpec((B, C), lambda i: (i, 0)),
            pl.BlockSpec((B, C), lambda i: (i, 0)),
            pl.BlockSpec((B, 1), lambda i: (i, 0)),
        ],
        out_specs=pl.BlockSpec((B, C), lambda i: (i, 0)),
        out_shape=jax.ShapeDtypeStruct((N, C), jnp.float32),
    )(pred_logits, gt_score, lab2d)


# speedup vs baseline: 18.5662x; 6.5135x over previous
import functools

import jax
import jax.numpy as jnp
from jax import lax
from jax.experimental import pallas as pl


def _qfl_block_t(pred_ref, gts_ref, label_ref, out_ref):
    x = pred_ref[...]
    g = gts_ref[...]
    lab = label_ref[...]
    C, B = x.shape

    rows = lax.broadcasted_iota(jnp.int32, (C, B), 0)
    onehot = rows == lab
    gs = jnp.sum(jnp.where(onehot, g, 0.0), axis=0, keepdims=True)

    e = jnp.exp(-jnp.abs(x))
    r = 1.0 / (1.0 + e)
    s = jnp.where(x >= 0, r, e * r)
    sp = jnp.log(1.0 + e)
    relu_x = jnp.maximum(x, 0.0)
    ce = (relu_x + sp) * (s * s)

    sf = gs - s
    vals = (relu_x - x * gs + sp) * (sf * sf)

    mask = (lab >= 0) & (lab < C)
    out_ref[...] = jnp.where(onehot & mask, vals, ce)


@jax.jit
def kernel(pred_logits, gt_label, gt_score):
    N, C = pred_logits.shape
    BN = 12800
    grid = (pl.cdiv(N, BN),)
    pt = pred_logits.T
    gt = gt_score.T
    lab = gt_label.astype(jnp.int32).reshape(1, N)
    out_t = pl.pallas_call(
        _qfl_block_t,
        grid=grid,
        in_specs=[
            pl.BlockSpec((C, BN), lambda i: (0, i)),
            pl.BlockSpec((C, BN), lambda i: (0, i)),
            pl.BlockSpec((1, BN), lambda i: (0, i)),
        ],
        out_specs=pl.BlockSpec((C, BN), lambda i: (0, i)),
        out_shape=jax.ShapeDtypeStruct((C, N), jnp.float32),
    )(pt, gt, lab)
    return out_t.T


# speedup vs baseline: 18.9334x; 1.0198x over previous
import functools

import jax
import jax.numpy as jnp
from jax import lax
from jax.experimental import pallas as pl


def _qfl_block_t(pred_ref, gts_ref, label_ref, out_ref):
    x = pred_ref[...]
    g = gts_ref[...]
    lab = label_ref[...]
    C, B = x.shape

    rows = lax.broadcasted_iota(jnp.int32, (C, B), 0)
    onehot = rows == lab
    gs = jnp.sum(jnp.where(onehot, g, 0.0), axis=0, keepdims=True)

    e = jnp.exp(-jnp.abs(x))
    r = 1.0 / (1.0 + e)
    s = jnp.where(x >= 0, r, e * r)
    sp = jnp.log(1.0 + e)
    relu_x = jnp.maximum(x, 0.0)
    ce = (relu_x + sp) * (s * s)

    sf = gs - s
    vals = (relu_x - x * gs + sp) * (sf * sf)

    mask = (lab >= 0) & (lab < C)
    out_ref[...] = jnp.where(onehot & mask, vals, ce)


@jax.jit
def kernel(pred_logits, gt_label, gt_score):
    N, C = pred_logits.shape
    BN = 25600
    grid = (pl.cdiv(N, BN),)
    pt = pred_logits.T
    gt = gt_score.T
    lab = gt_label.astype(jnp.int32).reshape(1, N)
    out_t = pl.pallas_call(
        _qfl_block_t,
        grid=grid,
        in_specs=[
            pl.BlockSpec((C, BN), lambda i: (0, i)),
            pl.BlockSpec((C, BN), lambda i: (0, i)),
            pl.BlockSpec((1, BN), lambda i: (0, i)),
        ],
        out_specs=pl.BlockSpec((C, BN), lambda i: (0, i)),
        out_shape=jax.ShapeDtypeStruct((C, N), jnp.float32),
    )(pt, gt, lab)
    return out_t.T


# speedup vs baseline: 19.1194x; 1.0098x over previous
import functools

import jax
import jax.numpy as jnp
from jax import lax
from jax.experimental import pallas as pl


def _qfl_block_t(pred_ref, gts_ref, label_ref, out_ref):
    x = pred_ref[...]
    g = gts_ref[...]
    lab = label_ref[...]
    C, B = x.shape

    rows = lax.broadcasted_iota(jnp.int32, (C, B), 0)
    onehot = rows == lab
    gs = jnp.sum(jnp.where(onehot, g, 0.0), axis=0, keepdims=True)

    e = jnp.exp(-jnp.abs(x))
    r = 1.0 / (1.0 + e)
    s = jnp.where(x >= 0, r, e * r)
    sp = jnp.log(1.0 + e)
    relu_x = jnp.maximum(x, 0.0)
    ce = (relu_x + sp) * (s * s)

    sf = gs - s
    vals = (relu_x - x * gs + sp) * (sf * sf)

    mask = (lab >= 0) & (lab < C)
    out_ref[...] = jnp.where(onehot & mask, vals, ce)


@jax.jit
def kernel(pred_logits, gt_label, gt_score):
    N, C = pred_logits.shape
    BN = 25088
    grid = (pl.cdiv(N, BN),)
    pt = pred_logits.T
    gt = gt_score.T
    lab = gt_label.astype(jnp.int32).reshape(1, N)
    out_t = pl.pallas_call(
        _qfl_block_t,
        grid=grid,
        in_specs=[
            pl.BlockSpec((C, BN), lambda i: (0, i)),
            pl.BlockSpec((C, BN), lambda i: (0, i)),
            pl.BlockSpec((1, BN), lambda i: (0, i)),
        ],
        out_specs=pl.BlockSpec((C, BN), lambda i: (0, i)),
        out_shape=jax.ShapeDtypeStruct((C, N), jnp.float32),
    )(pt, gt, lab)
    return out_t.T


# speedup vs baseline: 20.0696x; 1.0497x over previous
import functools

import jax
import jax.numpy as jnp
from jax import lax
from jax.experimental import pallas as pl


def _qfl_block_t(pred_ref, gts_ref, label_ref, out_ref):
    x = pred_ref[...]
    g = gts_ref[...]
    lab = label_ref[...]
    C, B = x.shape

    rows = lax.broadcasted_iota(jnp.int32, (C, B), 0)
    onehot = rows == lab

    e = jnp.exp(-jnp.abs(x))
    r = 1.0 / (1.0 + e)
    s = jnp.where(x >= 0, r, e * r)
    sp = jnp.log(1.0 + e)
    base = jnp.maximum(x, 0.0) + sp

    ce = base * (s * s)
    diff = g - s
    vals = (base - x * g) * (diff * diff)

    out_ref[...] = jnp.where(onehot, vals, ce)


@jax.jit
def kernel(pred_logits, gt_label, gt_score):
    N, C = pred_logits.shape
    BN = 25088
    grid = (pl.cdiv(N, BN),)
    pt = pred_logits.T
    gt = gt_score.T
    lab = gt_label.astype(jnp.int32).reshape(1, N)
    out_t = pl.pallas_call(
        _qfl_block_t,
        grid=grid,
        in_specs=[
            pl.BlockSpec((C, BN), lambda i: (0, i)),
            pl.BlockSpec((C, BN), lambda i: (0, i)),
            pl.BlockSpec((1, BN), lambda i: (0, i)),
        ],
        out_specs=pl.BlockSpec((C, BN), lambda i: (0, i)),
        out_shape=jax.ShapeDtypeStruct((C, N), jnp.float32),
    )(pt, gt, lab)
    return out_t.T


# speedup vs baseline: 20.3134x; 1.0121x over previous
import functools

import jax
import jax.numpy as jnp
from jax import lax
from jax.experimental import pallas as pl
from jax.experimental.pallas import tpu as pltpu


def _qfl_block_t(pred_ref, gts_ref, label_ref, out_ref):
    x = pred_ref[...]
    g = gts_ref[...]
    lab = label_ref[...]
    C, B = x.shape

    rows = lax.broadcasted_iota(jnp.int32, (C, B), 0)
    onehot = rows == lab

    s = 0.5 * jnp.tanh(0.5 * x) + 0.5
    s_abs = jnp.where(x >= 0, s, 1.0 - s)
    sp = -jnp.log(s_abs)
    base = jnp.maximum(x, 0.0) + sp

    a = jnp.where(onehot, g, 0.0)
    t = jnp.where(onehot, g - s, s)
    out_ref[...] = (base - x * a) * (t * t)


@jax.jit
def kernel(pred_logits, gt_label, gt_score):
    N, C = pred_logits.shape
    BN = 25088
    grid = (pl.cdiv(N, BN),)
    pt = pred_logits.T
    gt = gt_score.T
    lab = gt_label.astype(jnp.int32).reshape(1, N)
    out_t = pl.pallas_call(
        _qfl_block_t,
        grid=grid,
        in_specs=[
            pl.BlockSpec((C, BN), lambda i: (0, i)),
            pl.BlockSpec((C, BN), lambda i: (0, i)),
            pl.BlockSpec((1, BN), lambda i: (0, i)),
        ],
        out_specs=pl.BlockSpec((C, BN), lambda i: (0, i)),
        out_shape=jax.ShapeDtypeStruct((C, N), jnp.float32),
    )(pt, gt, lab)
    return out_t.T
